# baseline (speedup 1.0000x reference)
import functools

import jax
import jax.numpy as jnp
from jax import lax
from jax.experimental import pallas as pl
from jax.experimental.pallas import tpu as pltpu
from jax.experimental.pallas import tpu_sc as plsc

NUM_CORES = 2
NUM_SUBCORES = 16
NUM_WORKERS = NUM_CORES * NUM_SUBCORES

BATCH = 4096
TEXT = 50
DIM = 128
NUM_SPLITS = 2
PART_B = BATCH // NUM_SPLITS
TC_BLOCK_B = 32


def _make_sc_gather(batch: int, text: int, dim: int, rows_per_chunk: int):
  assert batch % NUM_WORKERS == 0
  rows_per_w = batch // NUM_WORKERS
  assert rows_per_w % (2 * rows_per_chunk) == 0
  n_pairs = rows_per_w // (2 * rows_per_chunk)
  chunk = rows_per_chunk * text
  idx_per_w = rows_per_w * text
  assert chunk % 8 == 0

  mesh = plsc.VectorSubcoreMesh(core_axis_name="c", subcore_axis_name="s")

  @functools.partial(
      pl.kernel,
      mesh=mesh,
      out_type=jax.ShapeDtypeStruct((batch * text, dim), jnp.float32),
      scratch_types=[
          pltpu.VMEM((idx_per_w,), jnp.int32),
          pltpu.VMEM((chunk, dim), jnp.float32),
          pltpu.VMEM((chunk, dim), jnp.float32),
          pltpu.SemaphoreType.DMA,
          pltpu.SemaphoreType.DMA,
      ],
  )
  def sc_gather(table_hbm, idx_hbm, out_hbm, idx_v, buf0, buf1, sem0, sem1):
    wid = lax.axis_index("s") * NUM_CORES + lax.axis_index("c")
    base = wid * idx_per_w
    pltpu.sync_copy(idx_hbm.at[pl.ds(base, idx_per_w)], idx_v)

    def gather_start(c, buf, sem):
      pltpu.async_copy(
          table_hbm.at[idx_v.at[pl.ds(c * chunk, chunk)]], buf, sem
      )

    def gather_wait(c, buf, sem):
      pltpu.make_async_copy(
          table_hbm.at[idx_v.at[pl.ds(c * chunk, chunk)]], buf, sem
      ).wait()

    def store(c, buf):
      pltpu.sync_copy(buf, out_hbm.at[pl.ds(base + c * chunk, chunk)])

    gather_start(0, buf0, sem0)

    def body(p, carry):
      c0 = 2 * p
      gather_start(c0 + 1, buf1, sem1)
      gather_wait(c0, buf0, sem0)
      store(c0, buf0)

      @pl.when(p + 1 < n_pairs)
      def _():
        gather_start(c0 + 2, buf0, sem0)

      gather_wait(c0 + 1, buf1, sem1)
      store(c0 + 1, buf1)
      return carry

    lax.fori_loop(0, n_pairs, body, 0)

  return sc_gather


def _tc_relayout_body(flat_ref, out_ref):
  for r in range(TC_BLOCK_B):
    out_ref[r] = flat_ref[pl.ds(r * TEXT, TEXT), :]


def _make_tc_relayout(part: int, first: bool):
  grid = (PART_B // TC_BLOCK_B,)
  row0 = part * PART_B // TC_BLOCK_B

  in_specs = [
      pl.BlockSpec((TC_BLOCK_B * TEXT, DIM), lambda g: (g, 0)),
  ]
  out_spec = pl.BlockSpec((TC_BLOCK_B, TEXT, DIM), lambda g: (row0 + g, 0, 0))
  out_shape = jax.ShapeDtypeStruct((BATCH, TEXT, DIM), jnp.float32)

  if first:
    def body(flat_ref, out_ref):
      _tc_relayout_body(flat_ref, out_ref)

    return pl.pallas_call(
        body, grid=grid, in_specs=in_specs, out_specs=out_spec,
        out_shape=out_shape,
    )

  def body(flat_ref, acc_ref, out_ref):
    del acc_ref
    _tc_relayout_body(flat_ref, out_ref)

  return functools.partial(
      pl.pallas_call(
          body, grid=grid,
          in_specs=in_specs + [pl.BlockSpec(memory_space=pl.ANY)],
          out_specs=out_spec, out_shape=out_shape,
          input_output_aliases={1: 0},
      ),
  )


_sc_gather_part = _make_sc_gather(PART_B, TEXT, DIM, 8)
_tc_first = _make_tc_relayout(0, True)
_tc_rest = [_make_tc_relayout(k, False) for k in range(1, NUM_SPLITS)]


def kernel(input, table):
  idx = input.astype(jnp.int32)
  flats = [
      _sc_gather_part(table, idx[k * PART_B:(k + 1) * PART_B].reshape(-1))
      for k in range(NUM_SPLITS)
  ]
  out = _tc_first(flats[0])
  for k in range(1, NUM_SPLITS):
    out = _tc_rest[k - 1](flats[k], out)
  return out
